# baseline (speedup 1.0000x reference)
import functools
import jax
import jax.numpy as jnp
from jax import lax
from jax.experimental import pallas as pl
from jax.experimental.pallas import tpu as pltpu, tpu_sc as plsc

N_NODES = 10000
N_EDGES = 160000
MUL = 16
NUM_BASIS = 32
MAX_RADIUS = 2.5
NUM_NEIGHBORS = 16
HIDDEN = 32
SILU_NORM = 1.6790

NPAD = 10240
EPAD = 163840
NW = 32
PER_W = EPAD // NW
GCHUNK = 1024
EB = 2048


@functools.lru_cache(maxsize=None)
def _make_gather(D):
    mesh = plsc.VectorSubcoreMesh(core_axis_name="c", subcore_axis_name="s")

    @functools.partial(
        pl.kernel, mesh=mesh,
        compiler_params=pltpu.CompilerParams(use_tc_tiling_on_sc=False),
        out_type=jax.ShapeDtypeStruct((EPAD, D), jnp.float32),
        scratch_types=[
            pltpu.VMEM((GCHUNK,), jnp.int32),
            pltpu.VMEM((GCHUNK, D), jnp.float32),
            pltpu.SemaphoreType.DMA,
        ],
    )
    def gather_k(table_hbm, idx_hbm, out_hbm, idx_v, rows_v, sem):
        wid = lax.axis_index("s") * 2 + lax.axis_index("c")
        base = wid * PER_W

        def body(ci, _):
            off = base + ci * GCHUNK
            pltpu.sync_copy(idx_hbm.at[pl.ds(off, GCHUNK)], idx_v)
            pltpu.async_copy(table_hbm.at[idx_v], rows_v, sem).wait()
            pltpu.sync_copy(rows_v, out_hbm.at[pl.ds(off, GCHUNK)])
            return 0

        lax.fori_loop(0, PER_W // GCHUNK, body, 0)

    return gather_k


def _edge_dense_body(gsrc, gdst, wq, wsim, w1k, w2k, w1v, w2v,
                     logits_ref, values_ref):
    g = gsrc[...]
    xs = g[:, :16]
    ps = g[:, 16:19]
    pd = gdst[...][:, :3]
    vec = ps - pd
    sq = jnp.sum(vec * vec, axis=1, keepdims=True)
    pos_mask = sq > 0.0
    sq_safe = jnp.where(pos_mask, sq, 1.0)
    vlen = jnp.where(pos_mask, jnp.sqrt(sq_safe), 0.0)
    x_safe = jnp.where(pos_mask, vlen, 1.0)
    nvec = (lax.broadcasted_iota(jnp.int32, (1, NUM_BASIS), 1) + 1
            ).astype(jnp.float32)
    rad = (jnp.sqrt(2.0 / MAX_RADIUS)
           * jnp.sin(nvec * (jnp.pi / MAX_RADIUS) * x_safe) / x_safe)
    rmask = pos_mask & (vlen < MAX_RADIUS)
    rad = jnp.where(rmask, rad * (NUM_BASIS ** 0.5), 0.0)
    y = 10.0 * (1.0 - vlen / MAX_RADIUS)
    y_safe = jnp.where(y > 0, y, 1.0)
    cutoff = jnp.where(y > 0, jnp.exp(-1.0 / y_safe), 0.0)

    inv_sqrt_h = 1.0 / jnp.sqrt(jnp.float32(HIDDEN))
    inv_sqrt_b = 1.0 / jnp.sqrt(jnp.float32(NUM_BASIS))

    def radial(w1, w2):
        h = jnp.dot(rad, w1[...]) * inv_sqrt_b
        h = SILU_NORM * (h * jax.nn.sigmoid(h))
        return jnp.dot(h, w2[...]) * inv_sqrt_h

    def contract(w_edge):
        acc = xs[:, 0:1] * w_edge[:, 0:16]
        for u in range(1, MUL):
            acc = acc + xs[:, u:u + 1] * w_edge[:, u * 16:(u + 1) * 16]
        return acc * 0.25

    key = contract(radial(w1k, w2k))
    values = contract(radial(w1v, w2v))
    qs = jnp.dot(xs, wq[...]) * 0.25
    qw = jnp.dot(qs, wsim[...])
    sim = jnp.sum(qw * key, axis=1, keepdims=True) * (1.0 / MUL)
    logits_ref[...] = cutoff * sim
    values_ref[...] = values


def _edge_dense(gsrc, gdst, wq, wsim2d, w1k, w2k, w1v, w2v):
    nblk = EPAD // EB
    full = lambda i: (0, 0)
    out = pl.pallas_call(
        _edge_dense_body,
        grid=(nblk,),
        in_specs=[
            pl.BlockSpec((EB, 32), lambda i: (i, 0)),
            pl.BlockSpec((EB, 16), lambda i: (i, 0)),
            pl.BlockSpec((MUL, MUL), full),
            pl.BlockSpec((MUL, MUL), full),
            pl.BlockSpec((NUM_BASIS, HIDDEN), full),
            pl.BlockSpec((HIDDEN, MUL * MUL), full),
            pl.BlockSpec((NUM_BASIS, HIDDEN), full),
            pl.BlockSpec((HIDDEN, MUL * MUL), full),
        ],
        out_specs=[
            pl.BlockSpec((EB, 1), lambda i: (i, 0)),
            pl.BlockSpec((EB, 16), lambda i: (i, 0)),
        ],
        out_shape=[
            jax.ShapeDtypeStruct((EPAD, 1), jnp.float32),
            jax.ShapeDtypeStruct((EPAD, 16), jnp.float32),
        ],
    )(gsrc, gdst, wq, wsim2d, w1k, w2k, w1v, w2v)
    return out


def kernel(x, pos, edge_index, W_query, W_sim, W1k, W2k, W1v, W2v):
    src = edge_index[0]
    dst = edge_index[1]
    pad_e = EPAD - N_EDGES
    src_p = jnp.concatenate([src, jnp.full((pad_e,), NPAD - 1, jnp.int32)])
    dst_p = jnp.concatenate([dst, jnp.full((pad_e,), NPAD - 1, jnp.int32)])

    t_src = jnp.zeros((NPAD, 32), jnp.float32)
    t_src = t_src.at[:N_NODES, :16].set(x).at[:N_NODES, 16:19].set(pos)
    t_dst = jnp.zeros((NPAD, 16), jnp.float32)
    t_dst = t_dst.at[:N_NODES, :3].set(pos)

    gsrc = _make_gather(32)(t_src, src_p)
    gdst = _make_gather(16)(t_dst, dst_p)

    logits2d, values = _edge_dense(
        gsrc, gdst, W_query, W_sim[:, :, 0], W1k, W2k, W1v, W2v)
    logits = logits2d[:, 0]

    m = jax.ops.segment_max(logits, src_p, num_segments=NPAD)
    m = jnp.where(jnp.isfinite(m), m, 0.0)
    ex = jnp.exp(logits - m[src_p])
    s = jax.ops.segment_sum(ex, src_p, num_segments=NPAD)
    coeff = jnp.sqrt(ex) / jnp.sqrt(s[src_p])
    out = jax.ops.segment_sum(coeff[:, None] * values, dst_p, num_segments=NPAD)
    return out[:N_NODES] / NUM_NEIGHBORS


# speedup vs baseline: 2.5611x; 2.5611x over previous
import functools
import jax
import jax.numpy as jnp
from jax import lax
from jax.experimental import pallas as pl
from jax.experimental.pallas import tpu as pltpu, tpu_sc as plsc

N_NODES = 10000
N_EDGES = 160000
MUL = 16
NUM_BASIS = 32
MAX_RADIUS = 2.5
NUM_NEIGHBORS = 16
HIDDEN = 32
SILU_NORM = 1.6790

NPAD = 10240
EPAD = 163840
NW = 32
PER_W = EPAD // NW
GCHUNK = 1024
EB = 2048


@functools.lru_cache(maxsize=None)
def _make_gather(D):
    mesh = plsc.VectorSubcoreMesh(core_axis_name="c", subcore_axis_name="s")

    @functools.partial(
        pl.kernel, mesh=mesh,
        compiler_params=pltpu.CompilerParams(use_tc_tiling_on_sc=False, needs_layout_passes=False),
        out_type=jax.ShapeDtypeStruct((EPAD, D), jnp.float32),
        scratch_types=[
            pltpu.VMEM((GCHUNK,), jnp.int32),
            pltpu.VMEM((GCHUNK, D), jnp.float32),
            pltpu.SemaphoreType.DMA,
        ],
    )
    def gather_k(table_hbm, idx_hbm, out_hbm, idx_v, rows_v, sem):
        wid = lax.axis_index("s") * 2 + lax.axis_index("c")
        base = wid * PER_W

        def body(ci, _):
            off = base + ci * GCHUNK
            pltpu.sync_copy(idx_hbm.at[pl.ds(off, GCHUNK)], idx_v)
            pltpu.async_copy(table_hbm.at[idx_v], rows_v, sem).wait()
            pltpu.sync_copy(rows_v, out_hbm.at[pl.ds(off, GCHUNK)])
            return 0

        lax.fori_loop(0, PER_W // GCHUNK, body, 0)

    return gather_k


def _edge_dense_body(gsrc, gdst, wq, wsim, w1k, w2k, w1v, w2v,
                     logits_ref, values_ref):
    g = gsrc[...]
    xs = g[:, :16]
    ps = g[:, 16:19]
    pd = gdst[...][:, :3]
    vec = ps - pd
    sq = jnp.sum(vec * vec, axis=1, keepdims=True)
    pos_mask = sq > 0.0
    sq_safe = jnp.where(pos_mask, sq, 1.0)
    vlen = jnp.where(pos_mask, jnp.sqrt(sq_safe), 0.0)
    x_safe = jnp.where(pos_mask, vlen, 1.0)
    nvec = (lax.broadcasted_iota(jnp.int32, (1, NUM_BASIS), 1) + 1
            ).astype(jnp.float32)
    rad = (jnp.sqrt(2.0 / MAX_RADIUS)
           * jnp.sin(nvec * (jnp.pi / MAX_RADIUS) * x_safe) / x_safe)
    rmask = pos_mask & (vlen < MAX_RADIUS)
    rad = jnp.where(rmask, rad * (NUM_BASIS ** 0.5), 0.0)
    y = 10.0 * (1.0 - vlen / MAX_RADIUS)
    y_safe = jnp.where(y > 0, y, 1.0)
    cutoff = jnp.where(y > 0, jnp.exp(-1.0 / y_safe), 0.0)

    inv_sqrt_h = 1.0 / jnp.sqrt(jnp.float32(HIDDEN))
    inv_sqrt_b = 1.0 / jnp.sqrt(jnp.float32(NUM_BASIS))

    def radial(w1, w2):
        h = jnp.dot(rad, w1[...]) * inv_sqrt_b
        h = SILU_NORM * (h * jax.nn.sigmoid(h))
        return jnp.dot(h, w2[...]) * inv_sqrt_h

    def contract(w_edge):
        acc = xs[:, 0:1] * w_edge[:, 0:16]
        for u in range(1, MUL):
            acc = acc + xs[:, u:u + 1] * w_edge[:, u * 16:(u + 1) * 16]
        return acc * 0.25

    key = contract(radial(w1k, w2k))
    values = contract(radial(w1v, w2v))
    qs = jnp.dot(xs, wq[...]) * 0.25
    qw = jnp.dot(qs, wsim[...])
    sim = jnp.sum(qw * key, axis=1, keepdims=True) * (1.0 / MUL)
    logits_ref[...] = cutoff * sim
    values_ref[...] = values


def _edge_dense(gsrc, gdst, wq, wsim2d, w1k, w2k, w1v, w2v):
    nblk = EPAD // EB
    full = lambda i: (0, 0)
    out = pl.pallas_call(
        _edge_dense_body,
        grid=(nblk,),
        in_specs=[
            pl.BlockSpec((EB, 32), lambda i: (i, 0)),
            pl.BlockSpec((EB, 16), lambda i: (i, 0)),
            pl.BlockSpec((MUL, MUL), full),
            pl.BlockSpec((MUL, MUL), full),
            pl.BlockSpec((NUM_BASIS, HIDDEN), full),
            pl.BlockSpec((HIDDEN, MUL * MUL), full),
            pl.BlockSpec((NUM_BASIS, HIDDEN), full),
            pl.BlockSpec((HIDDEN, MUL * MUL), full),
        ],
        out_specs=[
            pl.BlockSpec((EB, 1), lambda i: (i, 0)),
            pl.BlockSpec((EB, 16), lambda i: (i, 0)),
        ],
        out_shape=[
            jax.ShapeDtypeStruct((EPAD, 1), jnp.float32),
            jax.ShapeDtypeStruct((EPAD, 16), jnp.float32),
        ],
    )(gsrc, gdst, wq, wsim2d, w1k, w2k, w1v, w2v)
    return out


NEG = -3.0e38
NSUB = 16
SL = NPAD // NSUB
VEC = 16


def _take(v, idx):
    return jnp.take_along_axis(v, idx, axis=0, mode="promise_in_bounds")


def _seg_total(k2, v2, is_max):
    iota = lax.broadcasted_iota(jnp.int32, (VEC,), 0)
    for sh in (1, 2, 4, 8):
        pidx = jnp.maximum(iota - sh, 0)
        kk = _take(k2, pidx)
        vv = _take(v2, pidx)
        valid = (iota >= sh) & (kk == k2)
        upd = jnp.maximum(v2, vv) if is_max else v2 + vv
        v2 = jnp.where(valid, upd, v2)
    for sh in (1, 2, 4, 8):
        nidx = jnp.minimum(iota + sh, VEC - 1)
        kk = _take(k2, nidx)
        vv = _take(v2, nidx)
        valid = (iota < VEC - sh) & (kk == k2)
        v2 = jnp.where(valid, jnp.maximum(v2, vv), v2)
    return v2


def _fill(ref, val):
    def body(i, _):
        ref[pl.ds(i * VEC, VEC)] = jnp.full((VEC,), val, jnp.float32)
        return 0
    lax.fori_loop(0, ref.shape[0] // VEC, body, 0)


def _merge_private(priv, shared, out_hbm, core, sid, acc, tmpm, is_max):
    pltpu.sync_copy(priv, shared.at[sid])
    plsc.subcore_barrier()
    pltpu.sync_copy(shared.at[0, pl.ds(sid * SL, SL)], acc)

    def outer(j, _):
        pltpu.sync_copy(shared.at[j, pl.ds(sid * SL, SL)], tmpm)

        def inner(i, _):
            a = acc[pl.ds(i * VEC, VEC)]
            b = tmpm[pl.ds(i * VEC, VEC)]
            acc[pl.ds(i * VEC, VEC)] = (
                jnp.maximum(a, b) if is_max else a + b)
            return 0

        lax.fori_loop(0, SL // VEC, inner, 0)
        return 0

    lax.fori_loop(1, NSUB, outer, 0)
    pltpu.sync_copy(acc, out_hbm.at[core, pl.ds(sid * SL, SL)])


def _load_two_combine(part_hbm, dst, tmpn, is_max):
    pltpu.sync_copy(part_hbm.at[0], dst)
    pltpu.sync_copy(part_hbm.at[1], tmpn)

    def body(i, _):
        a = dst[pl.ds(i * VEC, VEC)]
        b = tmpn[pl.ds(i * VEC, VEC)]
        dst[pl.ds(i * VEC, VEC)] = jnp.maximum(a, b) if is_max else a + b
        return 0

    lax.fori_loop(0, NPAD // VEC, body, 0)


def _rsqrt(x):
    i = lax.bitcast_convert_type(x, jnp.int32)
    i = 0x5F3759DF - lax.shift_right_arithmetic(i, 1)
    y = lax.bitcast_convert_type(i, jnp.float32)
    for _ in range(3):
        y = y * (1.5 - 0.5 * x * y * y)
    return y


def _sc_mesh():
    return plsc.VectorSubcoreMesh(core_axis_name="c", subcore_axis_name="s")


@functools.lru_cache(maxsize=None)
def _make_segmax():
    @functools.partial(
        pl.kernel, mesh=_sc_mesh(),
        compiler_params=pltpu.CompilerParams(use_tc_tiling_on_sc=False, needs_layout_passes=False),
        out_type=jax.ShapeDtypeStruct((2, NPAD), jnp.float32),
        scratch_types=[
            pltpu.VMEM((NPAD,), jnp.float32),
            pltpu.VMEM((PER_W,), jnp.int32),
            pltpu.VMEM((PER_W,), jnp.float32),
            pltpu.VMEM_SHARED((NSUB, NPAD), jnp.float32),
            pltpu.VMEM((SL,), jnp.float32),
            pltpu.VMEM((SL,), jnp.float32),
        ],
    )
    def segmax_k(lg_hbm, src_hbm, m_out, m_priv, idx_v, lg_v, shared, acc,
                 tmpm):
        c = lax.axis_index("c")
        sid = lax.axis_index("s")
        wid = sid * 2 + c
        base = wid * PER_W
        _fill(m_priv, NEG)
        pltpu.sync_copy(src_hbm.at[pl.ds(base, PER_W)], idx_v)
        pltpu.sync_copy(lg_hbm.at[pl.ds(base, PER_W)], lg_v)

        def body(i, _):
            idx = idx_v[pl.ds(i * VEC, VEC)]
            l = lg_v[pl.ds(i * VEC, VEC)]
            k2, v2 = plsc.sort_key_val(idx, l)
            tot = _seg_total(k2, v2, True)
            cur = plsc.load_gather(m_priv, [k2])
            plsc.store_scatter(m_priv, [k2], jnp.maximum(cur, tot))
            return 0

        lax.fori_loop(0, PER_W // VEC, body, 0)
        _merge_private(m_priv, shared, m_out, c, sid, acc, tmpm, True)

    return segmax_k


@functools.lru_cache(maxsize=None)
def _make_exsum():
    @functools.partial(
        pl.kernel, mesh=_sc_mesh(),
        compiler_params=pltpu.CompilerParams(use_tc_tiling_on_sc=False, needs_layout_passes=False),
        out_type=[
            jax.ShapeDtypeStruct((EPAD,), jnp.float32),
            jax.ShapeDtypeStruct((2, NPAD), jnp.float32),
        ],
        scratch_types=[
            pltpu.VMEM((NPAD,), jnp.float32),
            pltpu.VMEM((NPAD,), jnp.float32),
            pltpu.VMEM((NPAD,), jnp.float32),
            pltpu.VMEM((PER_W,), jnp.int32),
            pltpu.VMEM((PER_W,), jnp.float32),
            pltpu.VMEM((PER_W,), jnp.float32),
            pltpu.VMEM_SHARED((NSUB, NPAD), jnp.float32),
            pltpu.VMEM((SL,), jnp.float32),
            pltpu.VMEM((SL,), jnp.float32),
        ],
    )
    def exsum_k(lg_hbm, src_hbm, m_hbm, exh_out, s_out, m_full, tmpn,
                s_priv, idx_v, lg_v, exh_v, shared, acc, tmpm):
        c = lax.axis_index("c")
        sid = lax.axis_index("s")
        wid = sid * 2 + c
        base = wid * PER_W
        _load_two_combine(m_hbm, m_full, tmpn, True)
        _fill(s_priv, 0.0)
        pltpu.sync_copy(src_hbm.at[pl.ds(base, PER_W)], idx_v)
        pltpu.sync_copy(lg_hbm.at[pl.ds(base, PER_W)], lg_v)

        def body(i, _):
            idx = idx_v[pl.ds(i * VEC, VEC)]
            l = lg_v[pl.ds(i * VEC, VEC)]
            msrc = plsc.load_gather(m_full, [idx])
            eh = jnp.exp(0.5 * (l - msrc))
            exh_v[pl.ds(i * VEC, VEC)] = eh
            k2, v2 = plsc.sort_key_val(idx, eh * eh)
            tot = _seg_total(k2, v2, False)
            cur = plsc.load_gather(s_priv, [k2])
            plsc.store_scatter(s_priv, [k2], cur + tot)
            return 0

        lax.fori_loop(0, PER_W // VEC, body, 0)
        pltpu.sync_copy(exh_v, exh_out.at[pl.ds(base, PER_W)])
        _merge_private(s_priv, shared, s_out, c, sid, acc, tmpm, False)

    return exsum_k


@functools.lru_cache(maxsize=None)
def _make_coeff():
    @functools.partial(
        pl.kernel, mesh=_sc_mesh(),
        compiler_params=pltpu.CompilerParams(use_tc_tiling_on_sc=False, needs_layout_passes=False),
        out_type=jax.ShapeDtypeStruct((EPAD,), jnp.float32),
        scratch_types=[
            pltpu.VMEM((NPAD,), jnp.float32),
            pltpu.VMEM((NPAD,), jnp.float32),
            pltpu.VMEM((PER_W,), jnp.int32),
            pltpu.VMEM((PER_W,), jnp.float32),
            pltpu.VMEM((PER_W,), jnp.float32),
        ],
    )
    def coeff_k(exh_hbm, src_hbm, s_hbm, co_out, s_full, tmpn, idx_v,
                exh_v, co_v):
        c = lax.axis_index("c")
        sid = lax.axis_index("s")
        wid = sid * 2 + c
        base = wid * PER_W
        _load_two_combine(s_hbm, s_full, tmpn, False)
        pltpu.sync_copy(src_hbm.at[pl.ds(base, PER_W)], idx_v)
        pltpu.sync_copy(exh_hbm.at[pl.ds(base, PER_W)], exh_v)

        def body(i, _):
            idx = idx_v[pl.ds(i * VEC, VEC)]
            eh = exh_v[pl.ds(i * VEC, VEC)]
            sv = plsc.load_gather(s_full, [idx])
            co_v[pl.ds(i * VEC, VEC)] = eh * _rsqrt(sv)
            return 0

        lax.fori_loop(0, PER_W // VEC, body, 0)
        pltpu.sync_copy(co_v, co_out.at[pl.ds(base, PER_W)])

    return coeff_k


SCH = 512


@functools.lru_cache(maxsize=None)
def _make_scatter_out():
    @functools.partial(
        pl.kernel, mesh=_sc_mesh(),
        compiler_params=pltpu.CompilerParams(use_tc_tiling_on_sc=False, needs_layout_passes=False),
        out_type=jax.ShapeDtypeStruct((2, NPAD, 16), jnp.float32),
        scratch_types=[
            pltpu.VMEM((SCH,), jnp.int32),
            pltpu.VMEM((SCH, 16), jnp.float32),
            pltpu.VMEM_SHARED((NPAD, 16), jnp.float32),
        ],
    )
    def scat_k(sv_hbm, dst_hbm, zero_hbm, out_hbm, di_v, rows, out_acc):
        c = lax.axis_index("c")
        sid = lax.axis_index("s")
        wid = sid * 2 + c
        base = wid * PER_W
        pltpu.sync_copy(zero_hbm.at[pl.ds(sid * SL, SL)],
                        out_acc.at[pl.ds(sid * SL, SL)])
        plsc.subcore_barrier()

        def body(ci, _):
            off = base + ci * SCH
            pltpu.sync_copy(dst_hbm.at[pl.ds(off, SCH)], di_v)
            pltpu.sync_copy(sv_hbm.at[pl.ds(off, SCH)], rows)
            pltpu.sync_copy(rows, out_acc.at[di_v], add=True)
            return 0

        lax.fori_loop(0, PER_W // SCH, body, 0)
        plsc.subcore_barrier()
        pltpu.sync_copy(out_acc.at[pl.ds(sid * SL, SL)],
                        out_hbm.at[c, pl.ds(sid * SL, SL)])

    return scat_k


def _scale_body(co_ref, val_ref, out_ref):
    out_ref[...] = co_ref[...] * val_ref[...] * (1.0 / NUM_NEIGHBORS)


def _combine_body(a_ref, b_ref, out_ref):
    out_ref[...] = a_ref[...] + b_ref[...]


def _scale_values(co, values):
    nb = EPAD // 4096
    return pl.pallas_call(
        _scale_body,
        grid=(nb,),
        in_specs=[
            pl.BlockSpec((4096, 1), lambda i: (i, 0)),
            pl.BlockSpec((4096, 16), lambda i: (i, 0)),
        ],
        out_specs=pl.BlockSpec((4096, 16), lambda i: (i, 0)),
        out_shape=jax.ShapeDtypeStruct((EPAD, 16), jnp.float32),
    )(co[:, None], values)


def _combine_partials(parts):
    return pl.pallas_call(
        _combine_body,
        grid=(NPAD // 2048,),
        in_specs=[
            pl.BlockSpec((2048, 16), lambda i: (i, 0)),
            pl.BlockSpec((2048, 16), lambda i: (i, 0)),
        ],
        out_specs=pl.BlockSpec((2048, 16), lambda i: (i, 0)),
        out_shape=jax.ShapeDtypeStruct((NPAD, 16), jnp.float32),
    )(parts[0], parts[1])


def kernel(x, pos, edge_index, W_query, W_sim, W1k, W2k, W1v, W2v):
    src = edge_index[0]
    dst = edge_index[1]
    pad_e = EPAD - N_EDGES
    src_p = jnp.concatenate([src, jnp.full((pad_e,), NPAD - 1, jnp.int32)])
    dst_p = jnp.concatenate([dst, jnp.full((pad_e,), NPAD - 1, jnp.int32)])

    t_src = jnp.zeros((NPAD, 32), jnp.float32)
    t_src = t_src.at[:N_NODES, :16].set(x).at[:N_NODES, 16:19].set(pos)
    t_dst = jnp.zeros((NPAD, 16), jnp.float32)
    t_dst = t_dst.at[:N_NODES, :3].set(pos)

    gsrc = _make_gather(32)(t_src, src_p)
    gdst = _make_gather(16)(t_dst, dst_p)

    logits2d, values = _edge_dense(
        gsrc, gdst, W_query, W_sim[:, :, 0], W1k, W2k, W1v, W2v)
    logits = logits2d[:, 0]

    m_part = _make_segmax()(logits, src_p)
    exh, s_part = _make_exsum()(logits, src_p, m_part)
    co = _make_coeff()(exh, src_p, s_part)
    scaled = _scale_values(co, values)
    zeros = jnp.zeros((NPAD, 16), jnp.float32)
    out_part = _make_scatter_out()(scaled, dst_p, zeros)
    out = _combine_partials(out_part)
    return out[:N_NODES]


# speedup vs baseline: 3.4968x; 1.3654x over previous
import functools
import jax
import jax.numpy as jnp
from jax import lax
from jax.experimental import pallas as pl
from jax.experimental.pallas import tpu as pltpu, tpu_sc as plsc

N_NODES = 10000
N_EDGES = 160000
MUL = 16
NUM_BASIS = 32
MAX_RADIUS = 2.5
NUM_NEIGHBORS = 16
HIDDEN = 32
SILU_NORM = 1.6790

NPAD = 10240
EPAD = 163840
NW = 32
PER_W = EPAD // NW
GCHUNK = 1024
EB = 2048


@functools.lru_cache(maxsize=None)
def _make_gather(D):
    mesh = plsc.VectorSubcoreMesh(core_axis_name="c", subcore_axis_name="s")

    @functools.partial(
        pl.kernel, mesh=mesh,
        compiler_params=pltpu.CompilerParams(use_tc_tiling_on_sc=False, needs_layout_passes=False),
        out_type=jax.ShapeDtypeStruct((EPAD, D), jnp.float32),
        scratch_types=[
            pltpu.VMEM((GCHUNK,), jnp.int32),
            pltpu.VMEM((GCHUNK, D), jnp.float32),
            pltpu.SemaphoreType.DMA,
        ],
    )
    def gather_k(table_hbm, idx_hbm, out_hbm, idx_v, rows_v, sem):
        wid = lax.axis_index("s") * 2 + lax.axis_index("c")
        base = wid * PER_W

        def body(ci, _):
            off = base + ci * GCHUNK
            pltpu.sync_copy(idx_hbm.at[pl.ds(off, GCHUNK)], idx_v)
            pltpu.async_copy(table_hbm.at[idx_v], rows_v, sem).wait()
            pltpu.sync_copy(rows_v, out_hbm.at[pl.ds(off, GCHUNK)])
            return 0

        lax.fori_loop(0, PER_W // GCHUNK, body, 0)

    return gather_k


def _edge_dense_body(gsrc, gdst, wq, wsim, w1k, w2k, w1v, w2v,
                     logits_ref, values_ref):
    g = gsrc[...]
    xs = g[:, :16]
    ps = g[:, 16:19]
    pd = gdst[...][:, :3]
    vec = ps - pd
    sq = jnp.sum(vec * vec, axis=1, keepdims=True)
    pos_mask = sq > 0.0
    sq_safe = jnp.where(pos_mask, sq, 1.0)
    vlen = jnp.where(pos_mask, jnp.sqrt(sq_safe), 0.0)
    x_safe = jnp.where(pos_mask, vlen, 1.0)
    nvec = (lax.broadcasted_iota(jnp.int32, (1, NUM_BASIS), 1) + 1
            ).astype(jnp.float32)
    rad = (jnp.sqrt(2.0 / MAX_RADIUS)
           * jnp.sin(nvec * (jnp.pi / MAX_RADIUS) * x_safe) / x_safe)
    rmask = pos_mask & (vlen < MAX_RADIUS)
    rad = jnp.where(rmask, rad * (NUM_BASIS ** 0.5), 0.0)
    y = 10.0 * (1.0 - vlen / MAX_RADIUS)
    y_safe = jnp.where(y > 0, y, 1.0)
    cutoff = jnp.where(y > 0, jnp.exp(-1.0 / y_safe), 0.0)

    inv_sqrt_h = 1.0 / jnp.sqrt(jnp.float32(HIDDEN))
    inv_sqrt_b = 1.0 / jnp.sqrt(jnp.float32(NUM_BASIS))

    def radial(w1, w2):
        h = jnp.dot(rad, w1[...]) * inv_sqrt_b
        h = SILU_NORM * (h * jax.nn.sigmoid(h))
        return jnp.dot(h, w2[...]) * inv_sqrt_h

    jr = lax.broadcasted_iota(jnp.int32, (MUL, MUL * MUL), 1)
    ur = lax.broadcasted_iota(jnp.int32, (MUL, MUL * MUL), 0)
    rm = (jr // MUL == ur).astype(jnp.float32)
    js = lax.broadcasted_iota(jnp.int32, (MUL * MUL, MUL), 0)
    ws = lax.broadcasted_iota(jnp.int32, (MUL * MUL, MUL), 1)
    sm = (js % MUL == ws).astype(jnp.float32)
    xr = jnp.dot(xs, rm, precision=lax.Precision.HIGHEST)

    def contract(w_edge):
        return jnp.dot(xr * w_edge, sm,
                       precision=lax.Precision.HIGHEST) * 0.25

    key = contract(radial(w1k, w2k))
    values = contract(radial(w1v, w2v))
    qs = jnp.dot(xs, wq[...]) * 0.25
    qw = jnp.dot(qs, wsim[...])
    sim = jnp.sum(qw * key, axis=1, keepdims=True) * (1.0 / MUL)
    logits_ref[...] = cutoff * sim
    values_ref[...] = values


def _edge_dense(gsrc, gdst, wq, wsim2d, w1k, w2k, w1v, w2v):
    nblk = EPAD // EB
    full = lambda i: (0, 0)
    out = pl.pallas_call(
        _edge_dense_body,
        grid=(nblk,),
        in_specs=[
            pl.BlockSpec((EB, 32), lambda i: (i, 0)),
            pl.BlockSpec((EB, 16), lambda i: (i, 0)),
            pl.BlockSpec((MUL, MUL), full),
            pl.BlockSpec((MUL, MUL), full),
            pl.BlockSpec((NUM_BASIS, HIDDEN), full),
            pl.BlockSpec((HIDDEN, MUL * MUL), full),
            pl.BlockSpec((NUM_BASIS, HIDDEN), full),
            pl.BlockSpec((HIDDEN, MUL * MUL), full),
        ],
        out_specs=[
            pl.BlockSpec((EB, 1), lambda i: (i, 0)),
            pl.BlockSpec((EB, 16), lambda i: (i, 0)),
        ],
        out_shape=[
            jax.ShapeDtypeStruct((EPAD, 1), jnp.float32),
            jax.ShapeDtypeStruct((EPAD, 16), jnp.float32),
        ],
    )(gsrc, gdst, wq, wsim2d, w1k, w2k, w1v, w2v)
    return out


NEG = -3.0e38
NSUB = 16
SL = NPAD // NSUB
VEC = 16


def _take(v, idx):
    return jnp.take_along_axis(v, idx, axis=0, mode="promise_in_bounds")


def _seg_total(k2, v2, is_max):
    iota = lax.broadcasted_iota(jnp.int32, (VEC,), 0)
    for sh in (1, 2, 4, 8):
        pidx = jnp.maximum(iota - sh, 0)
        kk = _take(k2, pidx)
        vv = _take(v2, pidx)
        valid = (iota >= sh) & (kk == k2)
        upd = jnp.maximum(v2, vv) if is_max else v2 + vv
        v2 = jnp.where(valid, upd, v2)
    for sh in (1, 2, 4, 8):
        nidx = jnp.minimum(iota + sh, VEC - 1)
        kk = _take(k2, nidx)
        vv = _take(v2, nidx)
        valid = (iota < VEC - sh) & (kk == k2)
        v2 = jnp.where(valid, jnp.maximum(v2, vv), v2)
    return v2


def _fill(ref, val):
    def body(i, _):
        ref[pl.ds(i * VEC, VEC)] = jnp.full((VEC,), val, jnp.float32)
        return 0
    lax.fori_loop(0, ref.shape[0] // VEC, body, 0)


def _merge_private(priv, shared, out_hbm, core, sid, acc, tmpm, is_max):
    pltpu.sync_copy(priv, shared.at[sid])
    plsc.subcore_barrier()
    pltpu.sync_copy(shared.at[0, pl.ds(sid * SL, SL)], acc)

    def outer(j, _):
        pltpu.sync_copy(shared.at[j, pl.ds(sid * SL, SL)], tmpm)

        def inner(i, _):
            a = acc[pl.ds(i * VEC, VEC)]
            b = tmpm[pl.ds(i * VEC, VEC)]
            acc[pl.ds(i * VEC, VEC)] = (
                jnp.maximum(a, b) if is_max else a + b)
            return 0

        lax.fori_loop(0, SL // VEC, inner, 0)
        return 0

    lax.fori_loop(1, NSUB, outer, 0)
    pltpu.sync_copy(acc, out_hbm.at[core, pl.ds(sid * SL, SL)])


def _load_two_combine(part_hbm, dst, tmpn, is_max):
    pltpu.sync_copy(part_hbm.at[0], dst)
    pltpu.sync_copy(part_hbm.at[1], tmpn)

    def body(i, _):
        a = dst[pl.ds(i * VEC, VEC)]
        b = tmpn[pl.ds(i * VEC, VEC)]
        dst[pl.ds(i * VEC, VEC)] = jnp.maximum(a, b) if is_max else a + b
        return 0

    lax.fori_loop(0, NPAD // VEC, body, 0)


def _rsqrt(x):
    i = lax.bitcast_convert_type(x, jnp.int32)
    i = 0x5F3759DF - lax.shift_right_arithmetic(i, 1)
    y = lax.bitcast_convert_type(i, jnp.float32)
    for _ in range(3):
        y = y * (1.5 - 0.5 * x * y * y)
    return y


def _sc_mesh():
    return plsc.VectorSubcoreMesh(core_axis_name="c", subcore_axis_name="s")


@functools.lru_cache(maxsize=None)
def _make_segmax():
    @functools.partial(
        pl.kernel, mesh=_sc_mesh(),
        compiler_params=pltpu.CompilerParams(use_tc_tiling_on_sc=False, needs_layout_passes=False),
        out_type=jax.ShapeDtypeStruct((2, NPAD), jnp.float32),
        scratch_types=[
            pltpu.VMEM((NPAD,), jnp.float32),
            pltpu.VMEM((PER_W,), jnp.int32),
            pltpu.VMEM((PER_W,), jnp.float32),
            pltpu.VMEM_SHARED((NSUB, NPAD), jnp.float32),
            pltpu.VMEM((SL,), jnp.float32),
            pltpu.VMEM((SL,), jnp.float32),
        ],
    )
    def segmax_k(lg_hbm, src_hbm, m_out, m_priv, idx_v, lg_v, shared, acc,
                 tmpm):
        c = lax.axis_index("c")
        sid = lax.axis_index("s")
        wid = sid * 2 + c
        base = wid * PER_W
        _fill(m_priv, NEG)
        pltpu.sync_copy(src_hbm.at[pl.ds(base, PER_W)], idx_v)
        pltpu.sync_copy(lg_hbm.at[pl.ds(base, PER_W)], lg_v)

        def body(i, _):
            idx = idx_v[pl.ds(i * VEC, VEC)]
            l = lg_v[pl.ds(i * VEC, VEC)]
            k2, v2 = plsc.sort_key_val(idx, l)
            tot = _seg_total(k2, v2, True)
            cur = plsc.load_gather(m_priv, [k2])
            plsc.store_scatter(m_priv, [k2], jnp.maximum(cur, tot))
            return 0

        lax.fori_loop(0, PER_W // VEC, body, 0)
        _merge_private(m_priv, shared, m_out, c, sid, acc, tmpm, True)

    return segmax_k


@functools.lru_cache(maxsize=None)
def _make_exsum():
    @functools.partial(
        pl.kernel, mesh=_sc_mesh(),
        compiler_params=pltpu.CompilerParams(use_tc_tiling_on_sc=False, needs_layout_passes=False),
        out_type=[
            jax.ShapeDtypeStruct((EPAD,), jnp.float32),
            jax.ShapeDtypeStruct((2, NPAD), jnp.float32),
        ],
        scratch_types=[
            pltpu.VMEM((NPAD,), jnp.float32),
            pltpu.VMEM((NPAD,), jnp.float32),
            pltpu.VMEM((NPAD,), jnp.float32),
            pltpu.VMEM((PER_W,), jnp.int32),
            pltpu.VMEM((PER_W,), jnp.float32),
            pltpu.VMEM((PER_W,), jnp.float32),
            pltpu.VMEM_SHARED((NSUB, NPAD), jnp.float32),
            pltpu.VMEM((SL,), jnp.float32),
            pltpu.VMEM((SL,), jnp.float32),
        ],
    )
    def exsum_k(lg_hbm, src_hbm, m_hbm, exh_out, s_out, m_full, tmpn,
                s_priv, idx_v, lg_v, exh_v, shared, acc, tmpm):
        c = lax.axis_index("c")
        sid = lax.axis_index("s")
        wid = sid * 2 + c
        base = wid * PER_W
        _load_two_combine(m_hbm, m_full, tmpn, True)
        _fill(s_priv, 0.0)
        pltpu.sync_copy(src_hbm.at[pl.ds(base, PER_W)], idx_v)
        pltpu.sync_copy(lg_hbm.at[pl.ds(base, PER_W)], lg_v)

        def body(i, _):
            idx = idx_v[pl.ds(i * VEC, VEC)]
            l = lg_v[pl.ds(i * VEC, VEC)]
            msrc = plsc.load_gather(m_full, [idx])
            eh = jnp.exp(0.5 * (l - msrc))
            exh_v[pl.ds(i * VEC, VEC)] = eh
            k2, v2 = plsc.sort_key_val(idx, eh * eh)
            tot = _seg_total(k2, v2, False)
            cur = plsc.load_gather(s_priv, [k2])
            plsc.store_scatter(s_priv, [k2], cur + tot)
            return 0

        lax.fori_loop(0, PER_W // VEC, body, 0)
        pltpu.sync_copy(exh_v, exh_out.at[pl.ds(base, PER_W)])
        _merge_private(s_priv, shared, s_out, c, sid, acc, tmpm, False)

    return exsum_k


@functools.lru_cache(maxsize=None)
def _make_coeff():
    @functools.partial(
        pl.kernel, mesh=_sc_mesh(),
        compiler_params=pltpu.CompilerParams(use_tc_tiling_on_sc=False, needs_layout_passes=False),
        out_type=jax.ShapeDtypeStruct((EPAD,), jnp.float32),
        scratch_types=[
            pltpu.VMEM((NPAD,), jnp.float32),
            pltpu.VMEM((NPAD,), jnp.float32),
            pltpu.VMEM((PER_W,), jnp.int32),
            pltpu.VMEM((PER_W,), jnp.float32),
            pltpu.VMEM((PER_W,), jnp.float32),
        ],
    )
    def coeff_k(exh_hbm, src_hbm, s_hbm, co_out, s_full, tmpn, idx_v,
                exh_v, co_v):
        c = lax.axis_index("c")
        sid = lax.axis_index("s")
        wid = sid * 2 + c
        base = wid * PER_W
        _load_two_combine(s_hbm, s_full, tmpn, False)
        pltpu.sync_copy(src_hbm.at[pl.ds(base, PER_W)], idx_v)
        pltpu.sync_copy(exh_hbm.at[pl.ds(base, PER_W)], exh_v)

        def body(i, _):
            idx = idx_v[pl.ds(i * VEC, VEC)]
            eh = exh_v[pl.ds(i * VEC, VEC)]
            sv = plsc.load_gather(s_full, [idx])
            co_v[pl.ds(i * VEC, VEC)] = eh * _rsqrt(sv)
            return 0

        lax.fori_loop(0, PER_W // VEC, body, 0)
        pltpu.sync_copy(co_v, co_out.at[pl.ds(base, PER_W)])

    return coeff_k


SCH = 512


@functools.lru_cache(maxsize=None)
def _make_scatter_out():
    @functools.partial(
        pl.kernel, mesh=_sc_mesh(),
        compiler_params=pltpu.CompilerParams(use_tc_tiling_on_sc=False, needs_layout_passes=False),
        out_type=jax.ShapeDtypeStruct((2, NPAD, 16), jnp.float32),
        scratch_types=[
            pltpu.VMEM((SCH,), jnp.int32),
            pltpu.VMEM((SCH, 16), jnp.float32),
            pltpu.VMEM_SHARED((NPAD, 16), jnp.float32),
        ],
    )
    def scat_k(sv_hbm, dst_hbm, zero_hbm, out_hbm, di_v, rows, out_acc):
        c = lax.axis_index("c")
        sid = lax.axis_index("s")
        wid = sid * 2 + c
        base = wid * PER_W
        pltpu.sync_copy(zero_hbm.at[pl.ds(sid * SL, SL)],
                        out_acc.at[pl.ds(sid * SL, SL)])
        plsc.subcore_barrier()

        def body(ci, _):
            off = base + ci * SCH
            pltpu.sync_copy(dst_hbm.at[pl.ds(off, SCH)], di_v)
            pltpu.sync_copy(sv_hbm.at[pl.ds(off, SCH)], rows)
            pltpu.sync_copy(rows, out_acc.at[di_v], add=True)
            return 0

        lax.fori_loop(0, PER_W // SCH, body, 0)
        plsc.subcore_barrier()
        pltpu.sync_copy(out_acc.at[pl.ds(sid * SL, SL)],
                        out_hbm.at[c, pl.ds(sid * SL, SL)])

    return scat_k


def _scale_body(co_ref, val_ref, out_ref):
    out_ref[...] = co_ref[...] * val_ref[...] * (1.0 / NUM_NEIGHBORS)


def _combine_body(a_ref, b_ref, out_ref):
    out_ref[...] = a_ref[...] + b_ref[...]


def _scale_values(co, values):
    nb = EPAD // 4096
    return pl.pallas_call(
        _scale_body,
        grid=(nb,),
        in_specs=[
            pl.BlockSpec((4096, 1), lambda i: (i, 0)),
            pl.BlockSpec((4096, 16), lambda i: (i, 0)),
        ],
        out_specs=pl.BlockSpec((4096, 16), lambda i: (i, 0)),
        out_shape=jax.ShapeDtypeStruct((EPAD, 16), jnp.float32),
    )(co[:, None], values)


def _combine_partials(parts):
    return pl.pallas_call(
        _combine_body,
        grid=(NPAD // 2048,),
        in_specs=[
            pl.BlockSpec((2048, 16), lambda i: (i, 0)),
            pl.BlockSpec((2048, 16), lambda i: (i, 0)),
        ],
        out_specs=pl.BlockSpec((2048, 16), lambda i: (i, 0)),
        out_shape=jax.ShapeDtypeStruct((NPAD, 16), jnp.float32),
    )(parts[0], parts[1])


def kernel(x, pos, edge_index, W_query, W_sim, W1k, W2k, W1v, W2v):
    src = edge_index[0]
    dst = edge_index[1]
    pad_e = EPAD - N_EDGES
    src_p = jnp.concatenate([src, jnp.full((pad_e,), NPAD - 1, jnp.int32)])
    dst_p = jnp.concatenate([dst, jnp.full((pad_e,), NPAD - 1, jnp.int32)])

    t_src = jnp.zeros((NPAD, 32), jnp.float32)
    t_src = t_src.at[:N_NODES, :16].set(x).at[:N_NODES, 16:19].set(pos)
    t_dst = jnp.zeros((NPAD, 16), jnp.float32)
    t_dst = t_dst.at[:N_NODES, :3].set(pos)

    gsrc = _make_gather(32)(t_src, src_p)
    gdst = _make_gather(16)(t_dst, dst_p)

    logits2d, values = _edge_dense(
        gsrc, gdst, W_query, W_sim[:, :, 0], W1k, W2k, W1v, W2v)
    logits = logits2d[:, 0]

    m_part = _make_segmax()(logits, src_p)
    exh, s_part = _make_exsum()(logits, src_p, m_part)
    co = _make_coeff()(exh, src_p, s_part)
    scaled = _scale_values(co, values)
    zeros = jnp.zeros((NPAD, 16), jnp.float32)
    out_part = _make_scatter_out()(scaled, dst_p, zeros)
    out = _combine_partials(out_part)
    return out[:N_NODES]


# speedup vs baseline: 3.8111x; 1.0899x over previous
import functools
import jax
import jax.numpy as jnp
from jax import lax
from jax.experimental import pallas as pl
from jax.experimental.pallas import tpu as pltpu, tpu_sc as plsc

N_NODES = 10000
N_EDGES = 160000
MUL = 16
NUM_BASIS = 32
MAX_RADIUS = 2.5
NUM_NEIGHBORS = 16
HIDDEN = 32
SILU_NORM = 1.6790

NPAD = 10240
EPAD = 163840
NW = 32
PER_W = EPAD // NW
GCHUNK = 1024
EB = 2048


@functools.lru_cache(maxsize=None)
def _make_gather(D):
    mesh = plsc.VectorSubcoreMesh(core_axis_name="c", subcore_axis_name="s")

    @functools.partial(
        pl.kernel, mesh=mesh,
        compiler_params=pltpu.CompilerParams(use_tc_tiling_on_sc=False, needs_layout_passes=False),
        out_type=jax.ShapeDtypeStruct((EPAD, D), jnp.float32),
        scratch_types=[
            pltpu.VMEM((GCHUNK,), jnp.int32),
            pltpu.VMEM((GCHUNK, D), jnp.float32),
            pltpu.SemaphoreType.DMA,
        ],
    )
    def gather_k(table_hbm, idx_hbm, out_hbm, idx_v, rows_v, sem):
        wid = lax.axis_index("s") * 2 + lax.axis_index("c")
        base = wid * PER_W

        def body(ci, _):
            off = base + ci * GCHUNK
            pltpu.sync_copy(idx_hbm.at[pl.ds(off, GCHUNK)], idx_v)
            pltpu.async_copy(table_hbm.at[idx_v], rows_v, sem).wait()
            pltpu.sync_copy(rows_v, out_hbm.at[pl.ds(off, GCHUNK)])
            return 0

        lax.fori_loop(0, PER_W // GCHUNK, body, 0)

    return gather_k


def _edge_dense_body(gsrc, gdst, wq, wsim, w1k, w2k, w1v, w2v,
                     logits_ref, values_ref):
    g = gsrc[...]
    xs = g[:, :16]
    ps = g[:, 16:19]
    pd = gdst[...][:, :3]
    vec = ps - pd
    sq = jnp.sum(vec * vec, axis=1, keepdims=True)
    pos_mask = sq > 0.0
    sq_safe = jnp.where(pos_mask, sq, 1.0)
    vlen = jnp.where(pos_mask, jnp.sqrt(sq_safe), 0.0)
    x_safe = jnp.where(pos_mask, vlen, 1.0)
    nvec = (lax.broadcasted_iota(jnp.int32, (1, NUM_BASIS), 1) + 1
            ).astype(jnp.float32)
    theta = nvec * (jnp.pi / MAX_RADIUS) * x_safe
    kf = jnp.round(theta * (1.0 / jnp.pi))
    r = (theta - kf * 3.140625) - kf * 9.6765358979324e-4
    sgn = jnp.where((kf.astype(jnp.int32) & 1) == 0, 1.0, -1.0)
    z = r * r
    p = -2.5052108385441720e-08
    p = p * z + 2.7557319223985893e-06
    p = p * z + -1.9841269841269841e-04
    p = p * z + 8.3333333333333332e-03
    p = p * z + -1.6666666666666666e-01
    p = p * z + 1.0
    sin_t = sgn * r * p
    rad = jnp.sqrt(2.0 / MAX_RADIUS) * sin_t / x_safe
    rmask = pos_mask & (vlen < MAX_RADIUS)
    rad = jnp.where(rmask, rad * (NUM_BASIS ** 0.5), 0.0)
    y = 10.0 * (1.0 - vlen / MAX_RADIUS)
    y_safe = jnp.where(y > 0, y, 1.0)
    cutoff = jnp.where(y > 0, jnp.exp(-1.0 / y_safe), 0.0)

    inv_sqrt_h = 1.0 / jnp.sqrt(jnp.float32(HIDDEN))
    inv_sqrt_b = 1.0 / jnp.sqrt(jnp.float32(NUM_BASIS))

    def radial(w1, w2):
        h = jnp.dot(rad, w1[...]) * inv_sqrt_b
        h = SILU_NORM * (h * jax.nn.sigmoid(h))
        return jnp.dot(h, w2[...]) * inv_sqrt_h

    jr = lax.broadcasted_iota(jnp.int32, (MUL, MUL * MUL), 1)
    ur = lax.broadcasted_iota(jnp.int32, (MUL, MUL * MUL), 0)
    rm = (jr // MUL == ur).astype(jnp.float32)
    js = lax.broadcasted_iota(jnp.int32, (MUL * MUL, MUL), 0)
    ws = lax.broadcasted_iota(jnp.int32, (MUL * MUL, MUL), 1)
    sm = (js % MUL == ws).astype(jnp.float32)
    xr = jnp.dot(xs, rm, precision=lax.Precision.HIGHEST)

    def contract(w_edge):
        return jnp.dot(xr * w_edge, sm,
                       precision=lax.Precision.HIGHEST) * 0.25

    key = contract(radial(w1k, w2k))
    values = contract(radial(w1v, w2v))
    qs = jnp.dot(xs, wq[...]) * 0.25
    qw = jnp.dot(qs, wsim[...])
    sim = jnp.sum(qw * key, axis=1, keepdims=True) * (1.0 / MUL)
    logits_ref[...] = cutoff * sim
    values_ref[...] = values


def _edge_dense(gsrc, gdst, wq, wsim2d, w1k, w2k, w1v, w2v):
    nblk = EPAD // EB
    full = lambda i: (0, 0)
    out = pl.pallas_call(
        _edge_dense_body,
        grid=(nblk,),
        in_specs=[
            pl.BlockSpec((EB, 32), lambda i: (i, 0)),
            pl.BlockSpec((EB, 16), lambda i: (i, 0)),
            pl.BlockSpec((MUL, MUL), full),
            pl.BlockSpec((MUL, MUL), full),
            pl.BlockSpec((NUM_BASIS, HIDDEN), full),
            pl.BlockSpec((HIDDEN, MUL * MUL), full),
            pl.BlockSpec((NUM_BASIS, HIDDEN), full),
            pl.BlockSpec((HIDDEN, MUL * MUL), full),
        ],
        out_specs=[
            pl.BlockSpec((EB, 1), lambda i: (i, 0)),
            pl.BlockSpec((EB, 16), lambda i: (i, 0)),
        ],
        out_shape=[
            jax.ShapeDtypeStruct((EPAD, 1), jnp.float32),
            jax.ShapeDtypeStruct((EPAD, 16), jnp.float32),
        ],
    )(gsrc, gdst, wq, wsim2d, w1k, w2k, w1v, w2v)
    return out


NEG = -3.0e38
NSUB = 16
SL = NPAD // NSUB
VEC = 16


def _take(v, idx):
    return jnp.take_along_axis(v, idx, axis=0, mode="promise_in_bounds")


def _seg_total(k2, v2, is_max):
    iota = lax.broadcasted_iota(jnp.int32, (VEC,), 0)
    for sh in (1, 2, 4, 8):
        pidx = jnp.maximum(iota - sh, 0)
        kk = _take(k2, pidx)
        vv = _take(v2, pidx)
        valid = (iota >= sh) & (kk == k2)
        upd = jnp.maximum(v2, vv) if is_max else v2 + vv
        v2 = jnp.where(valid, upd, v2)
    for sh in (1, 2, 4, 8):
        nidx = jnp.minimum(iota + sh, VEC - 1)
        kk = _take(k2, nidx)
        vv = _take(v2, nidx)
        valid = (iota < VEC - sh) & (kk == k2)
        v2 = jnp.where(valid, jnp.maximum(v2, vv), v2)
    return v2


def _fill(ref, val):
    def body(i, _):
        ref[pl.ds(i * VEC, VEC)] = jnp.full((VEC,), val, jnp.float32)
        return 0
    lax.fori_loop(0, ref.shape[0] // VEC, body, 0)


def _merge_private(priv, shared, out_hbm, core, sid, acc, tmpm, is_max):
    pltpu.sync_copy(priv, shared.at[sid])
    plsc.subcore_barrier()
    pltpu.sync_copy(shared.at[0, pl.ds(sid * SL, SL)], acc)

    def outer(j, _):
        pltpu.sync_copy(shared.at[j, pl.ds(sid * SL, SL)], tmpm)

        def inner(i, _):
            a = acc[pl.ds(i * VEC, VEC)]
            b = tmpm[pl.ds(i * VEC, VEC)]
            acc[pl.ds(i * VEC, VEC)] = (
                jnp.maximum(a, b) if is_max else a + b)
            return 0

        lax.fori_loop(0, SL // VEC, inner, 0)
        return 0

    lax.fori_loop(1, NSUB, outer, 0)
    pltpu.sync_copy(acc, out_hbm.at[core, pl.ds(sid * SL, SL)])


def _load_two_combine(part_hbm, dst, tmpn, is_max):
    pltpu.sync_copy(part_hbm.at[0], dst)
    pltpu.sync_copy(part_hbm.at[1], tmpn)

    def body(i, _):
        a = dst[pl.ds(i * VEC, VEC)]
        b = tmpn[pl.ds(i * VEC, VEC)]
        dst[pl.ds(i * VEC, VEC)] = jnp.maximum(a, b) if is_max else a + b
        return 0

    lax.fori_loop(0, NPAD // VEC, body, 0)


def _rsqrt(x):
    i = lax.bitcast_convert_type(x, jnp.int32)
    i = 0x5F3759DF - lax.shift_right_arithmetic(i, 1)
    y = lax.bitcast_convert_type(i, jnp.float32)
    for _ in range(3):
        y = y * (1.5 - 0.5 * x * y * y)
    return y


def _sc_mesh():
    return plsc.VectorSubcoreMesh(core_axis_name="c", subcore_axis_name="s")


@functools.lru_cache(maxsize=None)
def _make_segmax():
    @functools.partial(
        pl.kernel, mesh=_sc_mesh(),
        compiler_params=pltpu.CompilerParams(use_tc_tiling_on_sc=False, needs_layout_passes=False),
        out_type=jax.ShapeDtypeStruct((2, NPAD), jnp.float32),
        scratch_types=[
            pltpu.VMEM((NPAD,), jnp.float32),
            pltpu.VMEM((PER_W,), jnp.int32),
            pltpu.VMEM((PER_W,), jnp.float32),
            pltpu.VMEM_SHARED((NSUB, NPAD), jnp.float32),
            pltpu.VMEM((SL,), jnp.float32),
            pltpu.VMEM((SL,), jnp.float32),
        ],
    )
    def segmax_k(lg_hbm, src_hbm, m_out, m_priv, idx_v, lg_v, shared, acc,
                 tmpm):
        c = lax.axis_index("c")
        sid = lax.axis_index("s")
        wid = sid * 2 + c
        base = wid * PER_W
        _fill(m_priv, NEG)
        pltpu.sync_copy(src_hbm.at[pl.ds(base, PER_W)], idx_v)
        pltpu.sync_copy(lg_hbm.at[pl.ds(base, PER_W)], lg_v)

        def body(i, _):
            idx = idx_v[pl.ds(i * VEC, VEC)]
            l = lg_v[pl.ds(i * VEC, VEC)]
            k2, v2 = plsc.sort_key_val(idx, l)
            tot = _seg_total(k2, v2, True)
            cur = plsc.load_gather(m_priv, [k2])
            plsc.store_scatter(m_priv, [k2], jnp.maximum(cur, tot))
            return 0

        lax.fori_loop(0, PER_W // VEC, body, 0)
        _merge_private(m_priv, shared, m_out, c, sid, acc, tmpm, True)

    return segmax_k


@functools.lru_cache(maxsize=None)
def _make_exsum():
    @functools.partial(
        pl.kernel, mesh=_sc_mesh(),
        compiler_params=pltpu.CompilerParams(use_tc_tiling_on_sc=False, needs_layout_passes=False),
        out_type=[
            jax.ShapeDtypeStruct((EPAD,), jnp.float32),
            jax.ShapeDtypeStruct((2, NPAD), jnp.float32),
        ],
        scratch_types=[
            pltpu.VMEM((NPAD,), jnp.float32),
            pltpu.VMEM((NPAD,), jnp.float32),
            pltpu.VMEM((NPAD,), jnp.float32),
            pltpu.VMEM((PER_W,), jnp.int32),
            pltpu.VMEM((PER_W,), jnp.float32),
            pltpu.VMEM((PER_W,), jnp.float32),
            pltpu.VMEM_SHARED((NSUB, NPAD), jnp.float32),
            pltpu.VMEM((SL,), jnp.float32),
            pltpu.VMEM((SL,), jnp.float32),
        ],
    )
    def exsum_k(lg_hbm, src_hbm, m_hbm, exh_out, s_out, m_full, tmpn,
                s_priv, idx_v, lg_v, exh_v, shared, acc, tmpm):
        c = lax.axis_index("c")
        sid = lax.axis_index("s")
        wid = sid * 2 + c
        base = wid * PER_W
        _load_two_combine(m_hbm, m_full, tmpn, True)
        _fill(s_priv, 0.0)
        pltpu.sync_copy(src_hbm.at[pl.ds(base, PER_W)], idx_v)
        pltpu.sync_copy(lg_hbm.at[pl.ds(base, PER_W)], lg_v)

        def body(i, _):
            idx = idx_v[pl.ds(i * VEC, VEC)]
            l = lg_v[pl.ds(i * VEC, VEC)]
            msrc = plsc.load_gather(m_full, [idx])
            eh = jnp.exp(0.5 * (l - msrc))
            exh_v[pl.ds(i * VEC, VEC)] = eh
            k2, v2 = plsc.sort_key_val(idx, eh * eh)
            tot = _seg_total(k2, v2, False)
            cur = plsc.load_gather(s_priv, [k2])
            plsc.store_scatter(s_priv, [k2], cur + tot)
            return 0

        lax.fori_loop(0, PER_W // VEC, body, 0)
        pltpu.sync_copy(exh_v, exh_out.at[pl.ds(base, PER_W)])
        _merge_private(s_priv, shared, s_out, c, sid, acc, tmpm, False)

    return exsum_k


@functools.lru_cache(maxsize=None)
def _make_coeff():
    @functools.partial(
        pl.kernel, mesh=_sc_mesh(),
        compiler_params=pltpu.CompilerParams(use_tc_tiling_on_sc=False, needs_layout_passes=False),
        out_type=jax.ShapeDtypeStruct((EPAD,), jnp.float32),
        scratch_types=[
            pltpu.VMEM((NPAD,), jnp.float32),
            pltpu.VMEM((NPAD,), jnp.float32),
            pltpu.VMEM((PER_W,), jnp.int32),
            pltpu.VMEM((PER_W,), jnp.float32),
            pltpu.VMEM((PER_W,), jnp.float32),
        ],
    )
    def coeff_k(exh_hbm, src_hbm, s_hbm, co_out, s_full, tmpn, idx_v,
                exh_v, co_v):
        c = lax.axis_index("c")
        sid = lax.axis_index("s")
        wid = sid * 2 + c
        base = wid * PER_W
        _load_two_combine(s_hbm, s_full, tmpn, False)
        pltpu.sync_copy(src_hbm.at[pl.ds(base, PER_W)], idx_v)
        pltpu.sync_copy(exh_hbm.at[pl.ds(base, PER_W)], exh_v)

        def body(i, _):
            idx = idx_v[pl.ds(i * VEC, VEC)]
            eh = exh_v[pl.ds(i * VEC, VEC)]
            sv = plsc.load_gather(s_full, [idx])
            co_v[pl.ds(i * VEC, VEC)] = eh * _rsqrt(sv)
            return 0

        lax.fori_loop(0, PER_W // VEC, body, 0)
        pltpu.sync_copy(co_v, co_out.at[pl.ds(base, PER_W)])

    return coeff_k


SCH = 512


@functools.lru_cache(maxsize=None)
def _make_scatter_out():
    @functools.partial(
        pl.kernel, mesh=_sc_mesh(),
        compiler_params=pltpu.CompilerParams(use_tc_tiling_on_sc=False, needs_layout_passes=False),
        out_type=jax.ShapeDtypeStruct((2, NPAD, 16), jnp.float32),
        scratch_types=[
            pltpu.VMEM((SCH,), jnp.int32),
            pltpu.VMEM((SCH, 16), jnp.float32),
            pltpu.VMEM_SHARED((NPAD, 16), jnp.float32),
        ],
    )
    def scat_k(sv_hbm, dst_hbm, zero_hbm, out_hbm, di_v, rows, out_acc):
        c = lax.axis_index("c")
        sid = lax.axis_index("s")
        wid = sid * 2 + c
        base = wid * PER_W
        pltpu.sync_copy(zero_hbm.at[pl.ds(sid * SL, SL)],
                        out_acc.at[pl.ds(sid * SL, SL)])
        plsc.subcore_barrier()

        def body(ci, _):
            off = base + ci * SCH
            pltpu.sync_copy(dst_hbm.at[pl.ds(off, SCH)], di_v)
            pltpu.sync_copy(sv_hbm.at[pl.ds(off, SCH)], rows)
            pltpu.sync_copy(rows, out_acc.at[di_v], add=True)
            return 0

        lax.fori_loop(0, PER_W // SCH, body, 0)
        plsc.subcore_barrier()
        pltpu.sync_copy(out_acc.at[pl.ds(sid * SL, SL)],
                        out_hbm.at[c, pl.ds(sid * SL, SL)])

    return scat_k


def _scale_body(co_ref, val_ref, out_ref):
    out_ref[...] = co_ref[...] * val_ref[...] * (1.0 / NUM_NEIGHBORS)


def _combine_body(a_ref, b_ref, out_ref):
    out_ref[...] = a_ref[...] + b_ref[...]


def _scale_values(co, values):
    nb = EPAD // 4096
    return pl.pallas_call(
        _scale_body,
        grid=(nb,),
        in_specs=[
            pl.BlockSpec((4096, 1), lambda i: (i, 0)),
            pl.BlockSpec((4096, 16), lambda i: (i, 0)),
        ],
        out_specs=pl.BlockSpec((4096, 16), lambda i: (i, 0)),
        out_shape=jax.ShapeDtypeStruct((EPAD, 16), jnp.float32),
    )(co[:, None], values)


def _combine_partials(parts):
    return pl.pallas_call(
        _combine_body,
        grid=(NPAD // 2048,),
        in_specs=[
            pl.BlockSpec((2048, 16), lambda i: (i, 0)),
            pl.BlockSpec((2048, 16), lambda i: (i, 0)),
        ],
        out_specs=pl.BlockSpec((2048, 16), lambda i: (i, 0)),
        out_shape=jax.ShapeDtypeStruct((NPAD, 16), jnp.float32),
    )(parts[0], parts[1])


def kernel(x, pos, edge_index, W_query, W_sim, W1k, W2k, W1v, W2v):
    src = edge_index[0]
    dst = edge_index[1]
    pad_e = EPAD - N_EDGES
    src_p = jnp.concatenate([src, jnp.full((pad_e,), NPAD - 1, jnp.int32)])
    dst_p = jnp.concatenate([dst, jnp.full((pad_e,), NPAD - 1, jnp.int32)])

    t_src = jnp.zeros((NPAD, 32), jnp.float32)
    t_src = t_src.at[:N_NODES, :16].set(x).at[:N_NODES, 16:19].set(pos)
    t_dst = jnp.zeros((NPAD, 16), jnp.float32)
    t_dst = t_dst.at[:N_NODES, :3].set(pos)

    gsrc = _make_gather(32)(t_src, src_p)
    gdst = _make_gather(16)(t_dst, dst_p)

    logits2d, values = _edge_dense(
        gsrc, gdst, W_query, W_sim[:, :, 0], W1k, W2k, W1v, W2v)
    logits = logits2d[:, 0]

    m_part = _make_segmax()(logits, src_p)
    exh, s_part = _make_exsum()(logits, src_p, m_part)
    co = _make_coeff()(exh, src_p, s_part)
    scaled = _scale_values(co, values)
    zeros = jnp.zeros((NPAD, 16), jnp.float32)
    out_part = _make_scatter_out()(scaled, dst_p, zeros)
    out = _combine_partials(out_part)
    return out[:N_NODES]
